# Initial kernel scaffold; baseline (speedup 1.0000x reference)
import jax
import jax.numpy as jnp
from jax.experimental import pallas as pl


def kernel(xyz, x):
    raise NotImplementedError("write your pallas kernel here")


# speedup vs baseline: 7.6799x; 7.6799x over previous
import jax
import jax.numpy as jnp
from jax.experimental import pallas as pl
from jax.experimental.pallas import tpu as pltpu
from jax.experimental.pallas import tpu_sc as plsc

B, N, C = 4, 8192, 128
G = 2048
K = 32
TG = 256


def _fps_body(x0_ref, x1_ref, x2_ref, idx_ref, l0_ref, l1_ref, l2_ref):
    xa = x0_ref[...]
    xb = x1_ref[...]
    xc = x2_ref[...]
    iota = jax.lax.broadcasted_iota(jnp.int32, (B, N), 1)
    iota_g = jax.lax.broadcasted_iota(jnp.int32, (B, G), 1)
    base = jax.lax.broadcasted_iota(jnp.int32, (B, 1), 0) * N
    neg = jnp.float32(-jnp.inf)

    def body(i, carry):
        dists, far, accI, acc0, acc1, acc2 = carry
        mask = iota == far
        cx = jnp.max(jnp.where(mask, xa, neg), axis=1, keepdims=True)
        cy = jnp.max(jnp.where(mask, xb, neg), axis=1, keepdims=True)
        cz = jnp.max(jnp.where(mask, xc, neg), axis=1, keepdims=True)
        slot = iota_g == i
        accI = jnp.where(slot, far + base, accI)
        acc0 = jnp.where(slot, cx, acc0)
        acc1 = jnp.where(slot, cy, acc1)
        acc2 = jnp.where(slot, cz, acc2)
        dxa = xa - cx
        dxb = xb - cy
        dxc = xc - cz
        d = (dxa * dxa + dxc * dxc) + dxb * dxb
        dists = jnp.minimum(dists, d)
        m = jnp.max(dists, axis=1, keepdims=True)
        key = jnp.where(dists == m, iota, jnp.int32(N))
        far_new = jnp.min(key, axis=1, keepdims=True)
        return dists, far_new, accI, acc0, acc1, acc2

    carry0 = (
        jnp.full((B, N), 1e10, dtype=jnp.float32),
        jnp.zeros((B, 1), dtype=jnp.int32),
        jnp.zeros((B, G), dtype=jnp.int32),
        jnp.zeros((B, G), dtype=jnp.float32),
        jnp.zeros((B, G), dtype=jnp.float32),
        jnp.zeros((B, G), dtype=jnp.float32),
    )
    _, _, accI, acc0, acc1, acc2 = jax.lax.fori_loop(0, G, body, carry0)
    idx_ref[...] = accI
    l0_ref[...] = acc0
    l1_ref[...] = acc1
    l2_ref[...] = acc2


def _run_fps(x0, x1, x2):
    return pl.pallas_call(
        _fps_body,
        out_shape=(
            jax.ShapeDtypeStruct((B, G), jnp.int32),
            jax.ShapeDtypeStruct((B, G), jnp.float32),
            jax.ShapeDtypeStruct((B, G), jnp.float32),
            jax.ShapeDtypeStruct((B, G), jnp.float32),
        ),
    )(x0, x1, x2)


def _knn_body(l0_ref, l1_ref, l2_ref, x0_ref, x1_ref, x2_ref,
              out_ref, o0_ref, o1_ref, o2_ref):
    b = pl.program_id(0)
    q0 = l0_ref[...]
    q1 = l1_ref[...]
    q2 = l2_ref[...]
    p0 = x0_ref[0]
    p1 = x1_ref[0]
    p2 = x2_ref[0]

    s1 = (q0 * q0 + q1 * q1) + q2 * q2
    s2 = (p0 * p0 + p1 * p1) + p2 * p2

    qb0 = q0.astype(jnp.bfloat16).astype(jnp.float32)
    qb1 = q1.astype(jnp.bfloat16).astype(jnp.float32)
    qb2 = q2.astype(jnp.bfloat16).astype(jnp.float32)
    pb0 = p0.astype(jnp.bfloat16).astype(jnp.float32)
    pb1 = p1.astype(jnp.bfloat16).astype(jnp.float32)
    pb2 = p2.astype(jnp.bfloat16).astype(jnp.float32)
    t0 = qb0 * pb0
    t1 = qb1 * pb1
    t2 = qb2 * pb2
    s01 = t0 + t1
    bv = s01 - t0
    e01 = (t0 - (s01 - bv)) + (t1 - bv)
    s = s01 + t2
    bv2 = s - s01
    e2 = (s01 - (s - bv2)) + (t2 - bv2)
    mm = s + (e01 + e2)
    d = (jnp.float32(-2.0) * mm + s1) + s2

    iota = jax.lax.broadcasted_iota(jnp.int32, (TG, N), 1)
    boff = b * N
    neg = jnp.float32(-jnp.inf)
    for k in range(K):
        m = jnp.min(d, axis=1, keepdims=True)
        key = jnp.where(d == m, iota, jnp.int32(N))
        idx = jnp.min(key, axis=1, keepdims=True)
        out_ref[:, k:k + 1] = idx + boff
        sel = iota == idx
        o0_ref[:, k:k + 1] = jnp.max(jnp.where(sel, p0, neg), axis=1, keepdims=True)
        o1_ref[:, k:k + 1] = jnp.max(jnp.where(sel, p1, neg), axis=1, keepdims=True)
        o2_ref[:, k:k + 1] = jnp.max(jnp.where(sel, p2, neg), axis=1, keepdims=True)
        d = jnp.where(sel, jnp.float32(3e38), d)


def _run_knn(l0, l1, l2, x0, x1, x2):
    nt = G // TG
    qspec = pl.BlockSpec((TG, 1), lambda b, g: (b * nt + g, 0))
    pspec = pl.BlockSpec((1, 1, N), lambda b, g: (b, 0, 0))
    return pl.pallas_call(
        _knn_body,
        grid=(B, nt),
        in_specs=[qspec, qspec, qspec, pspec, pspec, pspec],
        out_specs=[pl.BlockSpec((TG, K), lambda b, g: (b * nt + g, 0))] * 4,
        out_shape=(
            jax.ShapeDtypeStruct((B * G, K), jnp.int32),
            jax.ShapeDtypeStruct((B * G, K), jnp.float32),
            jax.ShapeDtypeStruct((B * G, K), jnp.float32),
            jax.ShapeDtypeStruct((B * G, K), jnp.float32),
        ),
    )(l0, l1, l2, x0.reshape(B, 1, N), x1.reshape(B, 1, N), x2.reshape(B, 1, N))


def _sc_gather(table, idx_flat, window):
    m_idx = idx_flat.shape[1]
    vdim = table.shape[1]
    mesh = plsc.VectorSubcoreMesh(core_axis_name="c", subcore_axis_name="s")

    @pl.kernel(
        out_type=jax.ShapeDtypeStruct((m_idx, vdim), table.dtype),
        mesh=mesh,
    )
    def gk(tab_hbm, i_hbm, o_hbm):
        def body(i_vmem, o_vmem):
            pltpu.sync_copy(tab_hbm.at[i_vmem.at[0]], o_vmem)

        pltpu.emit_pipeline(
            body,
            grid=(m_idx // window,),
            in_specs=[pl.BlockSpec((1, window), lambda i: (0, i))],
            out_specs=[pl.BlockSpec((window, vdim), lambda i: (i, 0))],
            core_axis_name=("c", "s"),
            dimension_semantics=(pltpu.PARALLEL,),
        )(i_hbm, o_hbm)

    return gk(table, idx_flat)


def kernel(xyz, x):
    x0 = xyz[:, :, 0]
    x1 = xyz[:, :, 1]
    x2 = xyz[:, :, 2]

    fpsg, l0, l1, l2 = _run_fps(x0, x1, x2)
    lc_xyz = jnp.stack([l0, l1, l2], axis=-1)

    knn_gidx, o0, o1, o2 = _run_knn(
        l0.reshape(B * G, 1), l1.reshape(B * G, 1), l2.reshape(B * G, 1),
        x0, x1, x2,
    )

    x_flat = x.reshape(B * N, C)
    lc_x = _sc_gather(x_flat, fpsg.reshape(1, B * G), window=128)
    lc_x = lc_x.reshape(B, G, C)

    knn_idx_flat = knn_gidx.reshape(1, B * G * K)
    knn_x = _sc_gather(x_flat, knn_idx_flat, window=128)
    knn_x = knn_x.reshape(B, G, K, C)

    knn_xyz = jnp.stack([o0, o1, o2], axis=-1).reshape(B, G, K, 3)

    return (lc_xyz, lc_x, knn_xyz, knn_x)


# speedup vs baseline: 12.4371x; 1.6194x over previous
import jax
import jax.numpy as jnp
from jax.experimental import pallas as pl
from jax.experimental.pallas import tpu as pltpu
from jax.experimental.pallas import tpu_sc as plsc

B, N, C = 4, 8192, 128
G = 2048
K = 32
TG = 256


_FS = 8
_FL = N // _FS


def _fps_body(x0_ref, x1_ref, x2_ref, idx_ref, l0_ref, l1_ref, l2_ref):
    xa = x0_ref[...]
    xb = x1_ref[...]
    xc = x2_ref[...]
    iota = (jax.lax.broadcasted_iota(jnp.int32, (B, _FS, _FL), 1) * _FL
            + jax.lax.broadcasted_iota(jnp.int32, (B, _FS, _FL), 2))
    iota_g = jax.lax.broadcasted_iota(jnp.int32, (B, G), 1)
    base = jax.lax.broadcasted_iota(jnp.int32, (B, 1), 0) * N
    neg = jnp.float32(-jnp.inf)

    def body(i, carry):
        dists, far, accI, acc0, acc1, acc2 = carry
        mask = iota == far
        cx = jnp.max(jnp.where(mask, xa, neg), axis=(1, 2), keepdims=True)
        cy = jnp.max(jnp.where(mask, xb, neg), axis=(1, 2), keepdims=True)
        cz = jnp.max(jnp.where(mask, xc, neg), axis=(1, 2), keepdims=True)
        slot = iota_g == i
        accI = jnp.where(slot, far[:, :, 0] + base, accI)
        acc0 = jnp.where(slot, cx[:, :, 0], acc0)
        acc1 = jnp.where(slot, cy[:, :, 0], acc1)
        acc2 = jnp.where(slot, cz[:, :, 0], acc2)
        dxa = xa - cx
        dxb = xb - cy
        dxc = xc - cz
        d = (dxa * dxa + dxc * dxc) + dxb * dxb
        dists = jnp.minimum(dists, d)
        m = jnp.max(dists, axis=(1, 2), keepdims=True)
        key = jnp.where(dists == m, iota, jnp.int32(N))
        far_new = jnp.min(key, axis=(1, 2), keepdims=True)
        return dists, far_new, accI, acc0, acc1, acc2

    carry0 = (
        jnp.full((B, _FS, _FL), 1e10, dtype=jnp.float32),
        jnp.zeros((B, 1, 1), dtype=jnp.int32),
        jnp.zeros((B, G), dtype=jnp.int32),
        jnp.zeros((B, G), dtype=jnp.float32),
        jnp.zeros((B, G), dtype=jnp.float32),
        jnp.zeros((B, G), dtype=jnp.float32),
    )
    _, _, accI, acc0, acc1, acc2 = jax.lax.fori_loop(0, G, body, carry0)
    idx_ref[...] = accI
    l0_ref[...] = acc0
    l1_ref[...] = acc1
    l2_ref[...] = acc2


def _run_fps(x0, x1, x2):
    return pl.pallas_call(
        _fps_body,
        out_shape=(
            jax.ShapeDtypeStruct((B, G), jnp.int32),
            jax.ShapeDtypeStruct((B, G), jnp.float32),
            jax.ShapeDtypeStruct((B, G), jnp.float32),
            jax.ShapeDtypeStruct((B, G), jnp.float32),
        ),
    )(x0.reshape(B, _FS, _FL), x1.reshape(B, _FS, _FL), x2.reshape(B, _FS, _FL))


def _knn_body(l0_ref, l1_ref, l2_ref, x0_ref, x1_ref, x2_ref, out_ref):
    b = pl.program_id(0)
    q0 = l0_ref[...]
    q1 = l1_ref[...]
    q2 = l2_ref[...]
    p0 = x0_ref[0]
    p1 = x1_ref[0]
    p2 = x2_ref[0]

    s1 = (q0 * q0 + q1 * q1) + q2 * q2
    s2 = (p0 * p0 + p1 * p1) + p2 * p2

    qb0 = q0.astype(jnp.bfloat16).astype(jnp.float32)
    qb1 = q1.astype(jnp.bfloat16).astype(jnp.float32)
    qb2 = q2.astype(jnp.bfloat16).astype(jnp.float32)
    pb0 = p0.astype(jnp.bfloat16).astype(jnp.float32)
    pb1 = p1.astype(jnp.bfloat16).astype(jnp.float32)
    pb2 = p2.astype(jnp.bfloat16).astype(jnp.float32)
    t0 = qb0 * pb0
    t1 = qb1 * pb1
    t2 = qb2 * pb2
    s01 = t0 + t1
    bv = s01 - t0
    e01 = (t0 - (s01 - bv)) + (t1 - bv)
    s = s01 + t2
    bv2 = s - s01
    e2 = (s01 - (s - bv2)) + (t2 - bv2)
    mm = s + (e01 + e2)
    d = (jnp.float32(-2.0) * mm + s1) + s2

    iota = jax.lax.broadcasted_iota(jnp.int32, (TG, N), 1)
    boff = b * N
    for k in range(K):
        m = jnp.min(d, axis=1, keepdims=True)
        key = jnp.where(d == m, iota, jnp.int32(N))
        idx = jnp.min(key, axis=1, keepdims=True)
        out_ref[:, k:k + 1] = idx + boff
        d = jnp.where(iota == idx, jnp.float32(3e38), d)


def _run_knn(l0, l1, l2, x0, x1, x2):
    nt = G // TG
    qspec = pl.BlockSpec((TG, 1), lambda b, g: (b * nt + g, 0))
    pspec = pl.BlockSpec((1, 1, N), lambda b, g: (b, 0, 0))
    return pl.pallas_call(
        _knn_body,
        grid=(B, nt),
        in_specs=[qspec, qspec, qspec, pspec, pspec, pspec],
        out_specs=pl.BlockSpec((TG, K), lambda b, g: (b * nt + g, 0)),
        out_shape=jax.ShapeDtypeStruct((B * G, K), jnp.int32),
    )(l0, l1, l2, x0.reshape(B, 1, N), x1.reshape(B, 1, N), x2.reshape(B, 1, N))


def _sc_gather(table, idx_flat, window):
    m_idx = idx_flat.shape[1]
    vdim = table.shape[1]
    mesh = plsc.VectorSubcoreMesh(core_axis_name="c", subcore_axis_name="s")

    @pl.kernel(
        out_type=jax.ShapeDtypeStruct((m_idx, vdim), table.dtype),
        mesh=mesh,
    )
    def gk(tab_hbm, i_hbm, o_hbm):
        def body(i_vmem, o_vmem):
            pltpu.sync_copy(tab_hbm.at[i_vmem.at[0]], o_vmem)

        pltpu.emit_pipeline(
            body,
            grid=(m_idx // window,),
            in_specs=[pl.BlockSpec((1, window), lambda i: (0, i))],
            out_specs=[pl.BlockSpec((window, vdim), lambda i: (i, 0))],
            core_axis_name=("c", "s"),
            dimension_semantics=(pltpu.PARALLEL,),
        )(i_hbm, o_hbm)

    return gk(table, idx_flat)


def kernel(xyz, x):
    x0 = xyz[:, :, 0]
    x1 = xyz[:, :, 1]
    x2 = xyz[:, :, 2]

    fpsg, l0, l1, l2 = _run_fps(x0, x1, x2)
    lc_xyz = jnp.stack([l0, l1, l2], axis=-1)

    knn_gidx = _run_knn(
        l0.reshape(B * G, 1), l1.reshape(B * G, 1), l2.reshape(B * G, 1),
        x0, x1, x2,
    )

    x_flat = x.reshape(B * N, C)
    lc_x = _sc_gather(x_flat, fpsg.reshape(1, B * G), window=128)
    lc_x = lc_x.reshape(B, G, C)

    knn_idx_flat = knn_gidx.reshape(1, B * G * K)
    knn_x = _sc_gather(x_flat, knn_idx_flat, window=128)
    knn_x = knn_x.reshape(B, G, K, C)

    xyz_pad = jnp.pad(xyz.reshape(B * N, 3), ((0, 0), (0, 125)))
    knn_xyz = _sc_gather(xyz_pad, knn_idx_flat, window=128)
    knn_xyz = knn_xyz[:, :3].reshape(B, G, K, 3)

    return (lc_xyz, lc_x, knn_xyz, knn_x)
